# baseline (speedup 1.0000x reference)
import functools

import jax
import jax.numpy as jnp
from jax import lax
from jax.experimental import pallas as pl
from jax.experimental.pallas import tpu as pltpu
from jax.experimental.pallas import tpu_sc as plsc

N = 10000
E = 160000
D_IN = 32
D_EDGE = 16
HID = 32

NC = 2
NS = 16
NW = NC * NS
CH = 128
E_PAD = 163840
CHUNKS = E_PAD // (NW * CH)
EPW = E_PAD // NW
N_PAD = 10240
RPS = N_PAD // NS
CW = 16

BE = 512


def _mesh():
    return plsc.VectorSubcoreMesh(core_axis_name="c", subcore_axis_name="s")


_SC_PARAMS = pltpu.CompilerParams(use_tc_tiling_on_sc=False)


def _sc_gather(table, idx):
    @functools.partial(
        pl.kernel,
        out_type=jax.ShapeDtypeStruct((E_PAD, HID), jnp.float32),
        mesh=_mesh(),
        compiler_params=_SC_PARAMS,
        scratch_types=[
            pltpu.VMEM((CH,), jnp.int32),
            pltpu.VMEM((CH, HID), jnp.float32),
            pltpu.SemaphoreType.DMA,
        ],
    )
    def k(table_hbm, idx_hbm, out_hbm, idx_v, rows_v, sem):
        wid = lax.axis_index("c") * NS + lax.axis_index("s")
        base = wid * EPW

        def body(c, carry):
            off = pl.multiple_of(base + c * CH, CH)
            pltpu.sync_copy(idx_hbm.at[pl.ds(off, CH)], idx_v)
            pltpu.async_copy(table_hbm.at[idx_v], rows_v, sem).wait()
            pltpu.sync_copy(rows_v, out_hbm.at[pl.ds(off, CH)])
            return carry

        lax.fori_loop(0, CHUNKS, body, 0)

    return k(table, idx)


def _sc_scatter_counts(msg, dst, z32, z16, ones):
    @functools.partial(
        pl.kernel,
        out_type=[
            jax.ShapeDtypeStruct((N_PAD, HID), jnp.float32),
            jax.ShapeDtypeStruct((N_PAD, HID), jnp.float32),
            jax.ShapeDtypeStruct((N_PAD, CW), jnp.float32),
            jax.ShapeDtypeStruct((N_PAD, CW), jnp.float32),
        ],
        mesh=_mesh(),
        compiler_params=_SC_PARAMS,
        scratch_types=[
            pltpu.VMEM((CH,), jnp.int32),
            pltpu.VMEM((CH, HID), jnp.float32),
            pltpu.VMEM((CH, CW), jnp.float32),
            pltpu.VMEM_SHARED((N_PAD, HID), jnp.float32),
            pltpu.VMEM_SHARED((N_PAD, CW), jnp.float32),
        ],
    )
    def k(msg_hbm, dst_hbm, z32_hbm, z16_hbm, ones_hbm,
          acc0_hbm, acc1_hbm, cnt0_hbm, cnt1_hbm,
          idx_v, msg_v, ones_v, acc_sh, cnt_sh):
        cid = lax.axis_index("c")
        sid = lax.axis_index("s")
        wid = cid * NS + sid
        r0 = sid * RPS
        pltpu.sync_copy(z32_hbm.at[pl.ds(r0, RPS)], acc_sh.at[pl.ds(r0, RPS)])
        pltpu.sync_copy(z16_hbm.at[pl.ds(r0, RPS)], cnt_sh.at[pl.ds(r0, RPS)])
        pltpu.sync_copy(ones_hbm, ones_v)
        plsc.subcore_barrier()

        def body(c, carry):
            off = pl.multiple_of(wid * EPW + c * CH, CH)
            pltpu.sync_copy(dst_hbm.at[pl.ds(off, CH)], idx_v)
            pltpu.sync_copy(msg_hbm.at[pl.ds(off, CH)], msg_v)
            pltpu.sync_copy(msg_v, acc_sh.at[idx_v], add=True)
            pltpu.sync_copy(ones_v, cnt_sh.at[idx_v], add=True)
            return carry

        lax.fori_loop(0, CHUNKS, body, 0)
        plsc.subcore_barrier()

        @pl.when(cid == 0)
        def _():
            pltpu.sync_copy(acc_sh.at[pl.ds(r0, RPS)], acc0_hbm.at[pl.ds(r0, RPS)])
            pltpu.sync_copy(cnt_sh.at[pl.ds(r0, RPS)], cnt0_hbm.at[pl.ds(r0, RPS)])

        @pl.when(cid == 1)
        def _():
            pltpu.sync_copy(acc_sh.at[pl.ds(r0, RPS)], acc1_hbm.at[pl.ds(r0, RPS)])
            pltpu.sync_copy(cnt_sh.at[pl.ds(r0, RPS)], cnt1_hbm.at[pl.ds(r0, RPS)])

    return k(msg, dst, z32, z16, ones)


def _sc_scatter(msg, dst, z32):
    @functools.partial(
        pl.kernel,
        out_type=[
            jax.ShapeDtypeStruct((N_PAD, HID), jnp.float32),
            jax.ShapeDtypeStruct((N_PAD, HID), jnp.float32),
        ],
        mesh=_mesh(),
        compiler_params=_SC_PARAMS,
        scratch_types=[
            pltpu.VMEM((CH,), jnp.int32),
            pltpu.VMEM((CH, HID), jnp.float32),
            pltpu.VMEM_SHARED((N_PAD, HID), jnp.float32),
        ],
    )
    def k(msg_hbm, dst_hbm, z32_hbm, acc0_hbm, acc1_hbm,
          idx_v, msg_v, acc_sh):
        cid = lax.axis_index("c")
        sid = lax.axis_index("s")
        wid = cid * NS + sid
        r0 = sid * RPS
        pltpu.sync_copy(z32_hbm.at[pl.ds(r0, RPS)], acc_sh.at[pl.ds(r0, RPS)])
        plsc.subcore_barrier()

        def body(c, carry):
            off = pl.multiple_of(wid * EPW + c * CH, CH)
            pltpu.sync_copy(dst_hbm.at[pl.ds(off, CH)], idx_v)
            pltpu.sync_copy(msg_hbm.at[pl.ds(off, CH)], msg_v)
            pltpu.sync_copy(msg_v, acc_sh.at[idx_v], add=True)
            return carry

        lax.fori_loop(0, CHUNKS, body, 0)
        plsc.subcore_barrier()

        @pl.when(cid == 0)
        def _():
            pltpu.sync_copy(acc_sh.at[pl.ds(r0, RPS)], acc0_hbm.at[pl.ds(r0, RPS)])

        @pl.when(cid == 1)
        def _():
            pltpu.sync_copy(acc_sh.at[pl.ds(r0, RPS)], acc1_hbm.at[pl.ds(r0, RPS)])

    return k(msg, dst, z32)


def _msg_body(ea_ref, x_ref, A1_ref, b1_ref, A2m_ref, b2r_ref, out_ref):
    x = x_ref[...]
    h_e = jnp.maximum(
        jnp.dot(ea_ref[...], A1_ref[...], preferred_element_type=jnp.float32)
        + b1_ref[...], 0.0)
    u = jnp.repeat(h_e, HID, axis=1) * jnp.tile(x, (1, HID))
    acc = jnp.dot(u, A2m_ref[...], preferred_element_type=jnp.float32)
    acc = acc + jnp.dot(x, b2r_ref[...], preferred_element_type=jnp.float32)
    out_ref[...] = acc


def _tc_msg(ea, xsrc, A1, b1r, A2m, b2r):
    return pl.pallas_call(
        _msg_body,
        grid=(E_PAD // BE,),
        in_specs=[
            pl.BlockSpec((BE, D_EDGE), lambda i: (i, 0)),
            pl.BlockSpec((BE, D_IN), lambda i: (i, 0)),
            pl.BlockSpec((D_EDGE, HID), lambda i: (0, 0)),
            pl.BlockSpec((1, HID), lambda i: (0, 0)),
            pl.BlockSpec((D_IN * HID, HID), lambda i: (0, 0)),
            pl.BlockSpec((D_IN, HID), lambda i: (0, 0)),
        ],
        out_specs=pl.BlockSpec((BE, HID), lambda i: (i, 0)),
        out_shape=jax.ShapeDtypeStruct((E_PAD, HID), jnp.float32),
    )(ea, xsrc, A1, b1r, A2m, b2r)


def _layer_body(a0_ref, a1_ref, c0_ref, c1_ref, xin_ref, root_ref, bias_ref,
                out_ref):
    s = a0_ref[...] + a1_ref[...]
    c = c0_ref[:, 0:1] + c1_ref[:, 0:1]
    agg = s / jnp.maximum(c, 1.0)
    out_ref[...] = jnp.maximum(
        agg + jnp.dot(xin_ref[...], root_ref[...],
                      preferred_element_type=jnp.float32) + bias_ref[...],
        0.0)


def _tc_layer(a0, a1, c0, c1, xin, root, bias):
    return pl.pallas_call(
        _layer_body,
        out_shape=jax.ShapeDtypeStruct((N_PAD, HID), jnp.float32),
    )(a0, a1, c0, c1, xin, root, bias)


def _out_body(a0_ref, a1_ref, c0_ref, c1_ref, hin_ref, root_ref, bias_ref,
              wo_ref, bo_ref, out_ref):
    s = a0_ref[...] + a1_ref[...]
    c = c0_ref[:, 0:1] + c1_ref[:, 0:1]
    agg = s / jnp.maximum(c, 1.0)
    h2 = jnp.maximum(
        agg + jnp.dot(hin_ref[...], root_ref[...],
                      preferred_element_type=jnp.float32) + bias_ref[...],
        0.0)
    logit = jnp.sum(h2 * wo_ref[...], axis=1, keepdims=True) + bo_ref[...]
    out_ref[...] = jax.nn.sigmoid(logit)


def _tc_out(a0, a1, c0, c1, hin, root, bias, wo_row, bo):
    return pl.pallas_call(
        _out_body,
        out_shape=jax.ShapeDtypeStruct((N_PAD, 1), jnp.float32),
    )(a0, a1, c0, c1, hin, root, bias, wo_row, bo)


def kernel(x, edge_index, edge_attr, A1, b1, A2, b2, root1, bias1, root2,
           bias2, Wo, bo):
    src = edge_index[0]
    dst = edge_index[1]
    pad = E_PAD - E
    src_p = jnp.concatenate([src, jnp.zeros((pad,), jnp.int32)])
    dst_p = jnp.concatenate([dst, jnp.full((pad,), N, jnp.int32)])
    ea_p = jnp.concatenate([edge_attr, jnp.zeros((pad, D_EDGE), jnp.float32)])

    A2m = A2.reshape(HID * D_IN, HID)
    b2r = b2.reshape(D_IN, HID)
    b1r = b1.reshape(1, HID)
    xp = jnp.pad(x, ((0, N_PAD - N), (0, 0)))
    z32 = jnp.zeros((N_PAD, HID), jnp.float32)
    z16 = jnp.zeros((N_PAD, CW), jnp.float32)
    ones = jnp.ones((CH, CW), jnp.float32)

    xsrc = _sc_gather(xp, src_p)
    msg1 = _tc_msg(ea_p, xsrc, A1, b1r, A2m, b2r)
    a10, a11, c0, c1 = _sc_scatter_counts(msg1, dst_p, z32, z16, ones)
    h = _tc_layer(a10, a11, c0, c1, xp, root1, bias1.reshape(1, HID))

    hsrc = _sc_gather(h, src_p)
    msg2 = _tc_msg(ea_p, hsrc, A1, b1r, A2m, b2r)
    a20, a21 = _sc_scatter(msg2, dst_p, z32)
    out = _tc_out(a20, a21, c0, c1, h, root2, bias2.reshape(1, HID),
                  Wo.reshape(1, HID), bo.reshape(1, 1))
    return out[:N]


# speedup vs baseline: 2.2885x; 2.2885x over previous
import functools

import jax
import jax.numpy as jnp
from jax import lax
from jax.experimental import pallas as pl
from jax.experimental.pallas import tpu as pltpu
from jax.experimental.pallas import tpu_sc as plsc

N = 10000
E = 160000
D_IN = 32
D_EDGE = 16
HID = 32

NC = 2
NS = 16
NW = NC * NS
CH = 128
E_PAD = 163840
CHUNKS = E_PAD // (NW * CH)
EPW = E_PAD // NW
N_PAD = 10240
RPS = N_PAD // NS
CW = 16

BE = 512


def _mesh():
    return plsc.VectorSubcoreMesh(core_axis_name="c", subcore_axis_name="s")


_SC_PARAMS = pltpu.CompilerParams(use_tc_tiling_on_sc=False)


def _sc_gather(table, idx):
    @functools.partial(
        pl.kernel,
        out_type=jax.ShapeDtypeStruct((E_PAD, HID), jnp.float32),
        mesh=_mesh(),
        compiler_params=_SC_PARAMS,
        scratch_types=[
            pltpu.VMEM((CH,), jnp.int32),
            pltpu.VMEM((CH, HID), jnp.float32),
            pltpu.SemaphoreType.DMA,
        ],
    )
    def k(table_hbm, idx_hbm, out_hbm, idx_v, rows_v, sem):
        wid = lax.axis_index("c") * NS + lax.axis_index("s")
        base = wid * EPW

        def body(c, carry):
            off = pl.multiple_of(base + c * CH, CH)
            pltpu.sync_copy(idx_hbm.at[pl.ds(off, CH)], idx_v)
            pltpu.async_copy(table_hbm.at[idx_v], rows_v, sem).wait()
            pltpu.sync_copy(rows_v, out_hbm.at[pl.ds(off, CH)])
            return carry

        lax.fori_loop(0, CHUNKS, body, 0)

    return k(table, idx)


def _sc_scatter_counts(msg, dst, z32, z16, ones):
    @functools.partial(
        pl.kernel,
        out_type=[
            jax.ShapeDtypeStruct((N_PAD, HID), jnp.float32),
            jax.ShapeDtypeStruct((N_PAD, HID), jnp.float32),
            jax.ShapeDtypeStruct((N_PAD, CW), jnp.float32),
            jax.ShapeDtypeStruct((N_PAD, CW), jnp.float32),
        ],
        mesh=_mesh(),
        compiler_params=_SC_PARAMS,
        scratch_types=[
            pltpu.VMEM((CH,), jnp.int32),
            pltpu.VMEM((CH, HID), jnp.float32),
            pltpu.VMEM((CH, CW), jnp.float32),
            pltpu.VMEM_SHARED((N_PAD, HID), jnp.float32),
            pltpu.VMEM_SHARED((N_PAD, CW), jnp.float32),
        ],
    )
    def k(msg_hbm, dst_hbm, z32_hbm, z16_hbm, ones_hbm,
          acc0_hbm, acc1_hbm, cnt0_hbm, cnt1_hbm,
          idx_v, msg_v, ones_v, acc_sh, cnt_sh):
        cid = lax.axis_index("c")
        sid = lax.axis_index("s")
        wid = cid * NS + sid
        r0 = sid * RPS
        pltpu.sync_copy(z32_hbm.at[pl.ds(r0, RPS)], acc_sh.at[pl.ds(r0, RPS)])
        pltpu.sync_copy(z16_hbm.at[pl.ds(r0, RPS)], cnt_sh.at[pl.ds(r0, RPS)])
        pltpu.sync_copy(ones_hbm, ones_v)
        plsc.subcore_barrier()

        def body(c, carry):
            off = pl.multiple_of(wid * EPW + c * CH, CH)
            pltpu.sync_copy(dst_hbm.at[pl.ds(off, CH)], idx_v)
            pltpu.sync_copy(msg_hbm.at[pl.ds(off, CH)], msg_v)
            pltpu.sync_copy(msg_v, acc_sh.at[idx_v], add=True)
            pltpu.sync_copy(ones_v, cnt_sh.at[idx_v], add=True)
            return carry

        lax.fori_loop(0, CHUNKS, body, 0)
        plsc.subcore_barrier()

        @pl.when(cid == 0)
        def _():
            pltpu.sync_copy(acc_sh.at[pl.ds(r0, RPS)], acc0_hbm.at[pl.ds(r0, RPS)])
            pltpu.sync_copy(cnt_sh.at[pl.ds(r0, RPS)], cnt0_hbm.at[pl.ds(r0, RPS)])

        @pl.when(cid == 1)
        def _():
            pltpu.sync_copy(acc_sh.at[pl.ds(r0, RPS)], acc1_hbm.at[pl.ds(r0, RPS)])
            pltpu.sync_copy(cnt_sh.at[pl.ds(r0, RPS)], cnt1_hbm.at[pl.ds(r0, RPS)])

    return k(msg, dst, z32, z16, ones)


def _sc_scatter(msg, dst, z32):
    @functools.partial(
        pl.kernel,
        out_type=[
            jax.ShapeDtypeStruct((N_PAD, HID), jnp.float32),
            jax.ShapeDtypeStruct((N_PAD, HID), jnp.float32),
        ],
        mesh=_mesh(),
        compiler_params=_SC_PARAMS,
        scratch_types=[
            pltpu.VMEM((CH,), jnp.int32),
            pltpu.VMEM((CH, HID), jnp.float32),
            pltpu.VMEM_SHARED((N_PAD, HID), jnp.float32),
        ],
    )
    def k(msg_hbm, dst_hbm, z32_hbm, acc0_hbm, acc1_hbm,
          idx_v, msg_v, acc_sh):
        cid = lax.axis_index("c")
        sid = lax.axis_index("s")
        wid = cid * NS + sid
        r0 = sid * RPS
        pltpu.sync_copy(z32_hbm.at[pl.ds(r0, RPS)], acc_sh.at[pl.ds(r0, RPS)])
        plsc.subcore_barrier()

        def body(c, carry):
            off = pl.multiple_of(wid * EPW + c * CH, CH)
            pltpu.sync_copy(dst_hbm.at[pl.ds(off, CH)], idx_v)
            pltpu.sync_copy(msg_hbm.at[pl.ds(off, CH)], msg_v)
            pltpu.sync_copy(msg_v, acc_sh.at[idx_v], add=True)
            return carry

        lax.fori_loop(0, CHUNKS, body, 0)
        plsc.subcore_barrier()

        @pl.when(cid == 0)
        def _():
            pltpu.sync_copy(acc_sh.at[pl.ds(r0, RPS)], acc0_hbm.at[pl.ds(r0, RPS)])

        @pl.when(cid == 1)
        def _():
            pltpu.sync_copy(acc_sh.at[pl.ds(r0, RPS)], acc1_hbm.at[pl.ds(r0, RPS)])

    return k(msg, dst, z32)


def _msg_body(ea_ref, x_ref, A1_ref, b1_ref, A2_ref, b2_ref, RT_ref, S_ref,
              out_ref):
    h_e = jnp.maximum(
        jnp.dot(ea_ref[...], A1_ref[...], preferred_element_type=jnp.float32)
        + b1_ref[...], 0.0)
    W = jnp.dot(h_e, A2_ref[...], preferred_element_type=jnp.float32) \
        + b2_ref[...]
    xt = jnp.dot(x_ref[...], RT_ref[...], preferred_element_type=jnp.float32)
    out_ref[...] = jnp.dot(W * xt, S_ref[...],
                           preferred_element_type=jnp.float32)


def _tc_msg(ea, xsrc, A1, b1r, A2, b2r, RT, S):
    return pl.pallas_call(
        _msg_body,
        grid=(E_PAD // BE,),
        in_specs=[
            pl.BlockSpec((BE, D_EDGE), lambda i: (i, 0)),
            pl.BlockSpec((BE, D_IN), lambda i: (i, 0)),
            pl.BlockSpec((D_EDGE, HID), lambda i: (0, 0)),
            pl.BlockSpec((1, HID), lambda i: (0, 0)),
            pl.BlockSpec((HID, D_IN * HID), lambda i: (0, 0)),
            pl.BlockSpec((1, D_IN * HID), lambda i: (0, 0)),
            pl.BlockSpec((D_IN, D_IN * HID), lambda i: (0, 0)),
            pl.BlockSpec((D_IN * HID, HID), lambda i: (0, 0)),
        ],
        out_specs=pl.BlockSpec((BE, HID), lambda i: (i, 0)),
        out_shape=jax.ShapeDtypeStruct((E_PAD, HID), jnp.float32),
    )(ea, xsrc, A1, b1r, A2, b2r, RT, S)


def _layer_body(a0_ref, a1_ref, c0_ref, c1_ref, xin_ref, root_ref, bias_ref,
                out_ref):
    s = a0_ref[...] + a1_ref[...]
    c = c0_ref[:, 0:1] + c1_ref[:, 0:1]
    agg = s / jnp.maximum(c, 1.0)
    out_ref[...] = jnp.maximum(
        agg + jnp.dot(xin_ref[...], root_ref[...],
                      preferred_element_type=jnp.float32) + bias_ref[...],
        0.0)


def _tc_layer(a0, a1, c0, c1, xin, root, bias):
    return pl.pallas_call(
        _layer_body,
        out_shape=jax.ShapeDtypeStruct((N_PAD, HID), jnp.float32),
    )(a0, a1, c0, c1, xin, root, bias)


def _out_body(a0_ref, a1_ref, c0_ref, c1_ref, hin_ref, root_ref, bias_ref,
              wo_ref, bo_ref, out_ref):
    s = a0_ref[...] + a1_ref[...]
    c = c0_ref[:, 0:1] + c1_ref[:, 0:1]
    agg = s / jnp.maximum(c, 1.0)
    h2 = jnp.maximum(
        agg + jnp.dot(hin_ref[...], root_ref[...],
                      preferred_element_type=jnp.float32) + bias_ref[...],
        0.0)
    logit = jnp.sum(h2 * wo_ref[...], axis=1, keepdims=True) + bo_ref[...]
    out_ref[...] = jax.nn.sigmoid(logit)


def _tc_out(a0, a1, c0, c1, hin, root, bias, wo_row, bo):
    return pl.pallas_call(
        _out_body,
        out_shape=jax.ShapeDtypeStruct((N_PAD, 1), jnp.float32),
    )(a0, a1, c0, c1, hin, root, bias, wo_row, bo)


def kernel(x, edge_index, edge_attr, A1, b1, A2, b2, root1, bias1, root2,
           bias2, Wo, bo):
    src = edge_index[0]
    dst = edge_index[1]
    pad = E_PAD - E
    src_p = jnp.concatenate([src, jnp.zeros((pad,), jnp.int32)])
    dst_p = jnp.concatenate([dst, jnp.full((pad,), N, jnp.int32)])
    ea_p = jnp.concatenate([edge_attr, jnp.zeros((pad, D_EDGE), jnp.float32)])

    RT = jnp.kron(jnp.eye(D_IN, dtype=jnp.float32),
                  jnp.ones((1, HID), jnp.float32))
    S = jnp.kron(jnp.ones((D_IN, 1), jnp.float32),
                 jnp.eye(HID, dtype=jnp.float32))
    b2r = b2.reshape(1, D_IN * HID)
    b1r = b1.reshape(1, HID)
    xp = jnp.pad(x, ((0, N_PAD - N), (0, 0)))
    z32 = jnp.zeros((N_PAD, HID), jnp.float32)
    z16 = jnp.zeros((N_PAD, CW), jnp.float32)
    ones = jnp.ones((CH, CW), jnp.float32)

    xsrc = _sc_gather(xp, src_p)
    msg1 = _tc_msg(ea_p, xsrc, A1, b1r, A2, b2r, RT, S)
    a10, a11, c0, c1 = _sc_scatter_counts(msg1, dst_p, z32, z16, ones)
    h = _tc_layer(a10, a11, c0, c1, xp, root1, bias1.reshape(1, HID))

    hsrc = _sc_gather(h, src_p)
    msg2 = _tc_msg(ea_p, hsrc, A1, b1r, A2, b2r, RT, S)
    a20, a21 = _sc_scatter(msg2, dst_p, z32)
    out = _tc_out(a20, a21, c0, c1, h, root2, bias2.reshape(1, HID),
                  Wo.reshape(1, HID), bo.reshape(1, 1))
    return out[:N]


# speedup vs baseline: 2.5489x; 1.1138x over previous
import functools

import jax
import jax.numpy as jnp
from jax import lax
from jax.experimental import pallas as pl
from jax.experimental.pallas import tpu as pltpu
from jax.experimental.pallas import tpu_sc as plsc

N = 10000
E = 160000
D_IN = 32
D_EDGE = 16
HID = 32

NC = 2
NS = 16
NW = NC * NS
CH = 128
E_PAD = 163840
CHUNKS = E_PAD // (NW * CH)
EPW = E_PAD // NW
N_PAD = 10240
RPS = N_PAD // NS
CW = 16
NBUF = 8

BE = 512


def _mesh():
    return plsc.VectorSubcoreMesh(core_axis_name="c", subcore_axis_name="s")


_SC_PARAMS = pltpu.CompilerParams(use_tc_tiling_on_sc=False)


def _sc_gather(table, idx2d):
    NG = CHUNKS // NBUF

    @functools.partial(
        pl.kernel,
        out_type=jax.ShapeDtypeStruct((E_PAD, HID), jnp.float32),
        mesh=_mesh(),
        compiler_params=_SC_PARAMS,
        scratch_types=[
            pltpu.VMEM((CHUNKS, CH), jnp.int32),
            pltpu.VMEM((2 * NBUF, CH, HID), jnp.float32),
            pltpu.SemaphoreType.DMA,
            pltpu.SemaphoreType.DMA,
            pltpu.SemaphoreType.DMA,
            pltpu.SemaphoreType.DMA,
        ],
    )
    def k(table_hbm, idx_hbm, out_hbm, idx_v, rows_v, g0, g1, w0, w1):
        gsem = (g0, g1)
        wsem = (w0, w1)
        wid = lax.axis_index("c") * NS + lax.axis_index("s")
        row0 = wid * CHUNKS
        base = wid * EPW
        pltpu.sync_copy(idx_hbm.at[pl.ds(row0, CHUNKS)], idx_v)

        def fire_gathers(g):
            s = g % 2
            return [
                pltpu.async_copy(
                    table_hbm.at[idx_v.at[g * NBUF + b]],
                    rows_v.at[s * NBUF + b], gsem[s])
                for b in range(NBUF)
            ]

        gh = {0: fire_gathers(0)}
        wh = {}
        for g in range(NG):
            s = g % 2
            for h in gh[g]:
                h.wait()
            whs = []
            for b in range(NBUF):
                c = g * NBUF + b
                off = pl.multiple_of(base + c * CH, CH)
                whs.append(pltpu.async_copy(
                    rows_v.at[s * NBUF + b], out_hbm.at[pl.ds(off, CH)],
                    wsem[s]))
            wh[g] = whs
            if g + 1 < NG:
                if g - 1 >= 0:
                    for h in wh[g - 1]:
                        h.wait()
                gh[g + 1] = fire_gathers(g + 1)
        for h in wh[NG - 2]:
            h.wait()
        for h in wh[NG - 1]:
            h.wait()

    return k(table, idx2d)


def _sc_scatter_counts(msg, dst2d, z32, z16, ones):
    NG = CHUNKS // NBUF

    @functools.partial(
        pl.kernel,
        out_type=[
            jax.ShapeDtypeStruct((N_PAD, HID), jnp.float32),
            jax.ShapeDtypeStruct((N_PAD, HID), jnp.float32),
            jax.ShapeDtypeStruct((N_PAD, CW), jnp.float32),
            jax.ShapeDtypeStruct((N_PAD, CW), jnp.float32),
        ],
        mesh=_mesh(),
        compiler_params=_SC_PARAMS,
        scratch_types=[
            pltpu.VMEM((CHUNKS, CH), jnp.int32),
            pltpu.VMEM((2 * NBUF, CH, HID), jnp.float32),
            pltpu.VMEM((CH, CW), jnp.float32),
            pltpu.VMEM_SHARED((N_PAD, HID), jnp.float32),
            pltpu.VMEM_SHARED((N_PAD, CW), jnp.float32),
            pltpu.SemaphoreType.DMA,
            pltpu.SemaphoreType.DMA,
        ],
    )
    def k(msg_hbm, dst_hbm, z32_hbm, z16_hbm, ones_hbm,
          acc0_hbm, acc1_hbm, cnt0_hbm, cnt1_hbm,
          idx_v, msg_v, ones_v, acc_sh, cnt_sh, m0, m1):
        msem = (m0, m1)
        cid = lax.axis_index("c")
        sid = lax.axis_index("s")
        wid = cid * NS + sid
        r0 = sid * RPS
        base = wid * EPW
        pltpu.sync_copy(z32_hbm.at[pl.ds(r0, RPS)], acc_sh.at[pl.ds(r0, RPS)])
        pltpu.sync_copy(z16_hbm.at[pl.ds(r0, RPS)], cnt_sh.at[pl.ds(r0, RPS)])
        pltpu.sync_copy(ones_hbm, ones_v)
        pltpu.sync_copy(dst_hbm.at[pl.ds(wid * CHUNKS, CHUNKS)], idx_v)
        plsc.subcore_barrier()

        def fire_loads(g):
            s = g % 2
            return [
                pltpu.async_copy(
                    msg_hbm.at[pl.ds(
                        pl.multiple_of(base + (g * NBUF + b) * CH, CH), CH)],
                    msg_v.at[s * NBUF + b], msem[s])
                for b in range(NBUF)
            ]

        mh = {0: fire_loads(0)}
        for g in range(NG):
            s = g % 2
            if g + 1 < NG:
                mh[g + 1] = fire_loads(g + 1)
            for h in mh[g]:
                h.wait()
            for b in range(NBUF):
                c = g * NBUF + b
                pltpu.sync_copy(msg_v.at[s * NBUF + b], acc_sh.at[idx_v.at[c]],
                                add=True)
                pltpu.sync_copy(ones_v, cnt_sh.at[idx_v.at[c]], add=True)

        plsc.subcore_barrier()

        @pl.when(cid == 0)
        def _():
            pltpu.sync_copy(acc_sh.at[pl.ds(r0, RPS)], acc0_hbm.at[pl.ds(r0, RPS)])
            pltpu.sync_copy(cnt_sh.at[pl.ds(r0, RPS)], cnt0_hbm.at[pl.ds(r0, RPS)])

        @pl.when(cid == 1)
        def _():
            pltpu.sync_copy(acc_sh.at[pl.ds(r0, RPS)], acc1_hbm.at[pl.ds(r0, RPS)])
            pltpu.sync_copy(cnt_sh.at[pl.ds(r0, RPS)], cnt1_hbm.at[pl.ds(r0, RPS)])

    return k(msg, dst2d, z32, z16, ones)


def _sc_scatter(msg, dst2d, z32):
    NG = CHUNKS // NBUF

    @functools.partial(
        pl.kernel,
        out_type=[
            jax.ShapeDtypeStruct((N_PAD, HID), jnp.float32),
            jax.ShapeDtypeStruct((N_PAD, HID), jnp.float32),
        ],
        mesh=_mesh(),
        compiler_params=_SC_PARAMS,
        scratch_types=[
            pltpu.VMEM((CHUNKS, CH), jnp.int32),
            pltpu.VMEM((2 * NBUF, CH, HID), jnp.float32),
            pltpu.VMEM_SHARED((N_PAD, HID), jnp.float32),
            pltpu.SemaphoreType.DMA,
            pltpu.SemaphoreType.DMA,
        ],
    )
    def k(msg_hbm, dst_hbm, z32_hbm, acc0_hbm, acc1_hbm,
          idx_v, msg_v, acc_sh, m0, m1):
        msem = (m0, m1)
        cid = lax.axis_index("c")
        sid = lax.axis_index("s")
        wid = cid * NS + sid
        r0 = sid * RPS
        base = wid * EPW
        pltpu.sync_copy(z32_hbm.at[pl.ds(r0, RPS)], acc_sh.at[pl.ds(r0, RPS)])
        pltpu.sync_copy(dst_hbm.at[pl.ds(wid * CHUNKS, CHUNKS)], idx_v)
        plsc.subcore_barrier()

        def fire_loads(g):
            s = g % 2
            return [
                pltpu.async_copy(
                    msg_hbm.at[pl.ds(
                        pl.multiple_of(base + (g * NBUF + b) * CH, CH), CH)],
                    msg_v.at[s * NBUF + b], msem[s])
                for b in range(NBUF)
            ]

        mh = {0: fire_loads(0)}
        for g in range(NG):
            s = g % 2
            if g + 1 < NG:
                mh[g + 1] = fire_loads(g + 1)
            for h in mh[g]:
                h.wait()
            for b in range(NBUF):
                c = g * NBUF + b
                pltpu.sync_copy(msg_v.at[s * NBUF + b], acc_sh.at[idx_v.at[c]],
                                add=True)

        plsc.subcore_barrier()

        @pl.when(cid == 0)
        def _():
            pltpu.sync_copy(acc_sh.at[pl.ds(r0, RPS)], acc0_hbm.at[pl.ds(r0, RPS)])

        @pl.when(cid == 1)
        def _():
            pltpu.sync_copy(acc_sh.at[pl.ds(r0, RPS)], acc1_hbm.at[pl.ds(r0, RPS)])

    return k(msg, dst2d, z32)


def _msg_body(ea_ref, x_ref, A1_ref, b1_ref, A2_ref, b2_ref, RT_ref, S_ref,
              out_ref):
    h_e = jnp.maximum(
        jnp.dot(ea_ref[...], A1_ref[...], preferred_element_type=jnp.float32)
        + b1_ref[...], 0.0)
    W = jnp.dot(h_e, A2_ref[...], preferred_element_type=jnp.float32) \
        + b2_ref[...]
    xt = jnp.dot(x_ref[...], RT_ref[...], preferred_element_type=jnp.float32)
    out_ref[...] = jnp.dot(W * xt, S_ref[...],
                           preferred_element_type=jnp.float32)


def _tc_msg(ea, xsrc, A1, b1r, A2, b2r, RT, S):
    return pl.pallas_call(
        _msg_body,
        grid=(E_PAD // BE,),
        in_specs=[
            pl.BlockSpec((BE, D_EDGE), lambda i: (i, 0)),
            pl.BlockSpec((BE, D_IN), lambda i: (i, 0)),
            pl.BlockSpec((D_EDGE, HID), lambda i: (0, 0)),
            pl.BlockSpec((1, HID), lambda i: (0, 0)),
            pl.BlockSpec((HID, D_IN * HID), lambda i: (0, 0)),
            pl.BlockSpec((1, D_IN * HID), lambda i: (0, 0)),
            pl.BlockSpec((D_IN, D_IN * HID), lambda i: (0, 0)),
            pl.BlockSpec((D_IN * HID, HID), lambda i: (0, 0)),
        ],
        out_specs=pl.BlockSpec((BE, HID), lambda i: (i, 0)),
        out_shape=jax.ShapeDtypeStruct((E_PAD, HID), jnp.float32),
    )(ea, xsrc, A1, b1r, A2, b2r, RT, S)


def _layer_body(a0_ref, a1_ref, c0_ref, c1_ref, xin_ref, root_ref, bias_ref,
                out_ref):
    s = a0_ref[...] + a1_ref[...]
    c = c0_ref[:, 0:1] + c1_ref[:, 0:1]
    agg = s / jnp.maximum(c, 1.0)
    out_ref[...] = jnp.maximum(
        agg + jnp.dot(xin_ref[...], root_ref[...],
                      preferred_element_type=jnp.float32) + bias_ref[...],
        0.0)


def _tc_layer(a0, a1, c0, c1, xin, root, bias):
    return pl.pallas_call(
        _layer_body,
        out_shape=jax.ShapeDtypeStruct((N_PAD, HID), jnp.float32),
    )(a0, a1, c0, c1, xin, root, bias)


def _out_body(a0_ref, a1_ref, c0_ref, c1_ref, hin_ref, root_ref, bias_ref,
              wo_ref, bo_ref, out_ref):
    s = a0_ref[...] + a1_ref[...]
    c = c0_ref[:, 0:1] + c1_ref[:, 0:1]
    agg = s / jnp.maximum(c, 1.0)
    h2 = jnp.maximum(
        agg + jnp.dot(hin_ref[...], root_ref[...],
                      preferred_element_type=jnp.float32) + bias_ref[...],
        0.0)
    logit = jnp.sum(h2 * wo_ref[...], axis=1, keepdims=True) + bo_ref[...]
    out_ref[...] = jax.nn.sigmoid(logit)


def _tc_out(a0, a1, c0, c1, hin, root, bias, wo_row, bo):
    return pl.pallas_call(
        _out_body,
        out_shape=jax.ShapeDtypeStruct((N_PAD, 1), jnp.float32),
    )(a0, a1, c0, c1, hin, root, bias, wo_row, bo)


def kernel(x, edge_index, edge_attr, A1, b1, A2, b2, root1, bias1, root2,
           bias2, Wo, bo):
    src = edge_index[0]
    dst = edge_index[1]
    pad = E_PAD - E
    src_p = jnp.concatenate([src, jnp.zeros((pad,), jnp.int32)])
    dst_p = jnp.concatenate([dst, jnp.full((pad,), N, jnp.int32)])
    src_p = src_p.reshape(E_PAD // CH, CH)
    dst_p = dst_p.reshape(E_PAD // CH, CH)
    ea_p = jnp.concatenate([edge_attr, jnp.zeros((pad, D_EDGE), jnp.float32)])

    RT = jnp.kron(jnp.eye(D_IN, dtype=jnp.float32),
                  jnp.ones((1, HID), jnp.float32))
    S = jnp.kron(jnp.ones((D_IN, 1), jnp.float32),
                 jnp.eye(HID, dtype=jnp.float32))
    b2r = b2.reshape(1, D_IN * HID)
    b1r = b1.reshape(1, HID)
    xp = jnp.pad(x, ((0, N_PAD - N), (0, 0)))
    z32 = jnp.zeros((N_PAD, HID), jnp.float32)
    z16 = jnp.zeros((N_PAD, CW), jnp.float32)
    ones = jnp.ones((CH, CW), jnp.float32)

    xsrc = _sc_gather(xp, src_p)
    msg1 = _tc_msg(ea_p, xsrc, A1, b1r, A2, b2r, RT, S)
    a10, a11, c0, c1 = _sc_scatter_counts(msg1, dst_p, z32, z16, ones)
    h = _tc_layer(a10, a11, c0, c1, xp, root1, bias1.reshape(1, HID))

    hsrc = _sc_gather(h, src_p)
    msg2 = _tc_msg(ea_p, hsrc, A1, b1r, A2, b2r, RT, S)
    a20, a21 = _sc_scatter(msg2, dst_p, z32)
    out = _tc_out(a20, a21, c0, c1, h, root2, bias2.reshape(1, HID),
                  Wo.reshape(1, HID), bo.reshape(1, 1))
    return out[:N]
